# baseline (speedup 1.0000x reference)
import functools

import jax
import jax.numpy as jnp
from jax import lax
from jax.experimental import pallas as pl
from jax.experimental.pallas import tpu as pltpu
from jax.experimental.pallas import tpu_sc as plsc

N = 10000
E = 320000
D = 128
S = 64
STEPS = 8
DT = 0.125
EPS = 1e-12

NC = 2
NSUB = 16
NW = NC * NSUB
CH = 128
CPW = 80
NQ = 4
QC = CPW // NQ
EPAD = NW * CPW * CH
NOUT = 10240
RPT = NOUT // NSUB

RB = 1000
GRID = N // RB

A_ACC = (DT / 6.0, DT / 3.0, DT / 3.0, DT / 6.0)
A_Y = (DT / 2.0, DT / 2.0, DT, 0.0)


def _sc_agg(table, srcr, dstr):
    mesh = plsc.VectorSubcoreMesh(core_axis_name="c", subcore_axis_name="s")

    @functools.partial(
        pl.kernel,
        mesh=mesh,
        out_type=jax.ShapeDtypeStruct((NC, NOUT, S), jnp.float32),
        scratch_types=[
            pltpu.VMEM((2, QC, CH), jnp.int32),
            pltpu.VMEM((2, QC, CH), jnp.int32),
            pltpu.VMEM((CH, S), jnp.float32),
            pltpu.VMEM((CH, S), jnp.float32),
            pltpu.VMEM((CH, S), jnp.float32),
            pltpu.VMEM((CH, S), jnp.float32),
            pltpu.VMEM_SHARED((N, S), jnp.float32),
            pltpu.VMEM_SHARED((NOUT, S), jnp.float32),
            pltpu.SemaphoreType.DMA,
            pltpu.SemaphoreType.DMA,
            pltpu.SemaphoreType.DMA,
            pltpu.SemaphoreType.DMA,
            pltpu.SemaphoreType.DMA,
            pltpu.SemaphoreType.DMA,
            pltpu.SemaphoreType.DMA,
        ],
        compiler_params=pltpu.CompilerParams(use_tc_tiling_on_sc=False),
    )
    def k(table_hbm, src_hbm, dst_hbm, out_hbm, src_v, dst_v, rows0_v,
          rows1_v, rows2_v, rows3_v, table_sh, agg_sh, sem0, sem1, sem2,
          sem3, semi0, semi1, semz):
        c = lax.axis_index("c")
        s = lax.axis_index("s")
        wid = c * NSUB + s
        banks = (rows0_v, rows1_v, rows2_v, rows3_v)
        sems = (sem0, sem1, sem2, sem3)
        semi = (semi0, semi1)

        def idx_load(q):
            pltpu.async_copy(src_hbm.at[wid, q], src_v.at[q % 2],
                             semi[q % 2])
            pltpu.async_copy(dst_hbm.at[wid, q], dst_v.at[q % 2],
                             semi[q % 2])

        def idx_wait(q):
            pltpu.make_async_copy(src_hbm.at[wid, q], src_v.at[q % 2],
                                  semi[q % 2]).wait()
            pltpu.make_async_copy(dst_hbm.at[wid, q], dst_v.at[q % 2],
                                  semi[q % 2]).wait()

        idx_load(0)
        pltpu.async_copy(table_hbm.at[pl.ds(s * (N // NSUB), N // NSUB)],
                         table_sh.at[pl.ds(s * (N // NSUB), N // NSUB)],
                         semz)

        def zrow(i, carry):
            for g in range(S // 16):
                rows0_v[i, pl.ds(g * 16, 16)] = jnp.zeros((16,), jnp.float32)
            return carry

        lax.fori_loop(0, CH, zrow, 0)
        for q in range(RPT // CH):
            pltpu.async_copy(rows0_v, agg_sh.at[pl.ds(s * RPT + q * CH, CH)],
                             semz)
        pltpu.make_async_copy(table_hbm.at[pl.ds(s * (N // NSUB), N // NSUB)],
                              table_sh.at[pl.ds(s * (N // NSUB), N // NSUB)],
                              semz).wait()
        for q in range(RPT // CH):
            pltpu.make_async_copy(rows0_v,
                                  agg_sh.at[pl.ds(s * RPT + q * CH, CH)],
                                  semz).wait()
        plsc.subcore_barrier()

        def gath(ib, jq, p):
            pltpu.async_copy(table_sh.at[src_v.at[ib, jq]], banks[p],
                             sems[p])

        def gath_wait(ib, jq, p):
            pltpu.make_async_copy(table_sh.at[src_v.at[ib, jq]], banks[p],
                                  sems[p]).wait()

        def scat(ib, jq, p):
            pltpu.async_copy(banks[p], agg_sh.at[dst_v.at[ib, jq]], sems[p],
                             add=True)

        def scat_wait(ib, jq, p):
            pltpu.make_async_copy(banks[p], agg_sh.at[dst_v.at[ib, jq]],
                                  sems[p]).wait()

        for q in range(NQ):
            ib = q % 2
            if q + 1 < NQ:
                idx_load(q + 1)
            idx_wait(q)
            gath(ib, 0, 0)
            gath(ib, 1, 1)

            def qbody(t, carry, ib=ib):
                for p in range(4):
                    jq = 4 * t + p
                    gath_wait(ib, jq, p)
                    scat(ib, jq, p)

                    @pl.when(jq >= 2)
                    def _():
                        scat_wait(ib, jq - 2, (p + 2) % 4)

                    @pl.when(jq <= QC - 3)
                    def _():
                        gath(ib, jq + 2, (p + 2) % 4)

                return carry

            lax.fori_loop(0, QC // 4, qbody, 0)
            scat_wait(ib, QC - 2, (QC - 2) % 4)
            scat_wait(ib, QC - 1, (QC - 1) % 4)
        plsc.subcore_barrier()

        pltpu.sync_copy(agg_sh.at[pl.ds(s * RPT, RPT)],
                        out_hbm.at[c, pl.ds(s * RPT, RPT)])

    return k(table, srcr, dstr)


def _softmax(z):
    m = jnp.max(z, axis=-1, keepdims=True)
    ez = jnp.exp(z - m)
    return ez / jnp.sum(ez, axis=-1, keepdims=True)


def _enc_body(x_ref, we_ref, be_ref, y0_ref, p0_ref):
    enc = lax.dot_general(x_ref[...], we_ref[...], (((1,), (1,)), ((), ())),
                          preferred_element_type=jnp.float32) + be_ref[...]
    y0 = _softmax(enc)
    y0_ref[...] = y0
    p0_ref[...] = _softmax(y0)


def _tc_encode(x, W_enc, be1):
    return pl.pallas_call(
        _enc_body,
        grid=(GRID,),
        in_specs=[
            pl.BlockSpec((RB, D), lambda i: (i, 0)),
            pl.BlockSpec((S, D), lambda i: (0, 0)),
            pl.BlockSpec((1, S), lambda i: (0, 0)),
        ],
        out_specs=[pl.BlockSpec((RB, S), lambda i: (i, 0))] * 2,
        out_shape=[jax.ShapeDtypeStruct((N, S), jnp.float32)] * 2,
    )(x, W_enc, be1)


def _deg_body(degp_ref, out_ref):
    deg = degp_ref[0, :, 0:1] + degp_ref[1, :, 0:1]
    out_ref[...] = jnp.broadcast_to(jnp.maximum(deg, 1.0), out_ref.shape)


def _tc_degc(degp):
    return pl.pallas_call(
        _deg_body,
        grid=(GRID,),
        in_specs=[pl.BlockSpec((NC, RB, S), lambda i: (0, i, 0))],
        out_specs=pl.BlockSpec((RB, S), lambda i: (i, 0)),
        out_shape=jax.ShapeDtypeStruct((N, S), jnp.float32),
    )(degp)


def _post_body(a_acc, a_y, last, yb_ref, ya_ref, p_ref, agg_ref, deg_ref,
               wc_ref, bc_ref, ya2_ref, p2_ref):
    agg = agg_ref[0] + agg_ref[1]
    aggn = agg / deg_ref[...]
    grad = lax.dot_general(aggn, wc_ref[...], (((1,), (1,)), ((), ())),
                           preferred_element_type=jnp.float32) + bc_ref[...]
    k = jnp.maximum(p_ref[...], EPS) * grad
    k = k - jnp.mean(k, axis=-1, keepdims=True)
    ya2 = ya_ref[...] + a_acc * k
    z = ya2 if last else yb_ref[...] + a_y * k
    ya2_ref[...] = ya2
    p2_ref[...] = _softmax(z)


def _tc_post(yb, ya, p, aggp, degp, W_conv, bc1, stage):
    body = functools.partial(_post_body, A_ACC[stage], A_Y[stage], stage == 3)
    return pl.pallas_call(
        body,
        grid=(GRID,),
        in_specs=[
            pl.BlockSpec((RB, S), lambda i: (i, 0)),
            pl.BlockSpec((RB, S), lambda i: (i, 0)),
            pl.BlockSpec((RB, S), lambda i: (i, 0)),
            pl.BlockSpec((NC, RB, S), lambda i: (0, i, 0)),
            pl.BlockSpec((RB, S), lambda i: (i, 0)),
            pl.BlockSpec((S, S), lambda i: (0, 0)),
            pl.BlockSpec((1, S), lambda i: (0, 0)),
        ],
        out_specs=[pl.BlockSpec((RB, S), lambda i: (i, 0))] * 2,
        out_shape=[jax.ShapeDtypeStruct((N, S), jnp.float32)] * 2,
    )(yb, ya, p, aggp, degp, W_conv, bc1)


def _dec_body(y_ref, wd_ref, bd_ref, out_ref):
    out_ref[...] = lax.dot_general(
        y_ref[...], wd_ref[...], (((1,), (1,)), ((), ())),
        preferred_element_type=jnp.float32) + bd_ref[...]


def _tc_decode(y, W_dec, bd1):
    return pl.pallas_call(
        _dec_body,
        grid=(GRID,),
        in_specs=[
            pl.BlockSpec((RB, S), lambda i: (i, 0)),
            pl.BlockSpec((D, S), lambda i: (0, 0)),
            pl.BlockSpec((1, D), lambda i: (0, 0)),
        ],
        out_specs=pl.BlockSpec((RB, D), lambda i: (i, 0)),
        out_shape=jax.ShapeDtypeStruct((N, D), jnp.float32),
    )(y, W_dec, bd1)


def kernel(x, edge_index, W_enc, b_enc, W_dec, b_dec, W_conv, b_conv):
    src = edge_index[0]
    dst = edge_index[1]
    pad = EPAD - E
    iota = jnp.arange(pad, dtype=jnp.int32)
    srcr = jnp.concatenate([src, iota % N]).reshape(NW, NQ, QC, CH)
    dstr = jnp.concatenate([dst, N + iota % (NOUT - N)]).reshape(
        NW, NQ, QC, CH)
    be1 = b_enc.reshape(1, S)
    bc1 = b_conv.reshape(1, S)
    bd1 = b_dec.reshape(1, D)

    degc = _tc_degc(_sc_agg(jnp.ones((N, S), jnp.float32), srcr, dstr))
    y0, probs = _tc_encode(x, W_enc, be1)
    yb = y0
    ya = y0
    for _step in range(STEPS):
        for stage in range(4):
            aggp = _sc_agg(probs, srcr, dstr)
            ya, probs = _tc_post(yb, ya, probs, aggp, degc, W_conv, bc1,
                                 stage)
        yb = ya
    return _tc_decode(ya, W_dec, bd1)


# speedup vs baseline: 1.0333x; 1.0333x over previous
import functools

import jax
import jax.numpy as jnp
from jax import lax
from jax.experimental import pallas as pl
from jax.experimental.pallas import tpu as pltpu
from jax.experimental.pallas import tpu_sc as plsc

N = 10000
E = 320000
D = 128
S = 64
STEPS = 8
DT = 0.125
EPS = 1e-12

NC = 2
NSUB = 16
NW = NC * NSUB
CH = 128
CPW = 81
EPAD = NW * CPW * CH
NOUT = 10240
RPT = NOUT // NSUB

RB = 1000
GRID = N // RB

A_ACC = (DT / 6.0, DT / 3.0, DT / 3.0, DT / 6.0)
A_Y = (DT / 2.0, DT / 2.0, DT, 0.0)


def _sc_agg(table, srcr, dstr):
    mesh = plsc.VectorSubcoreMesh(core_axis_name="c", subcore_axis_name="s")

    @functools.partial(
        pl.kernel,
        mesh=mesh,
        out_type=jax.ShapeDtypeStruct((NC, NOUT, S), jnp.float32),
        scratch_types=[
            pltpu.VMEM((CPW, CH), jnp.int32),
            pltpu.VMEM((CPW, CH), jnp.int32),
            pltpu.VMEM((CH, S), jnp.float32),
            pltpu.VMEM((CH, S), jnp.float32),
            pltpu.VMEM((CH, S), jnp.float32),
            pltpu.VMEM_SHARED((N, S), jnp.float32),
            pltpu.VMEM_SHARED((NOUT, S), jnp.float32),
            pltpu.SemaphoreType.DMA,
            pltpu.SemaphoreType.DMA,
            pltpu.SemaphoreType.DMA,
        ],
        compiler_params=pltpu.CompilerParams(use_tc_tiling_on_sc=False),
    )
    def k(table_hbm, src_hbm, dst_hbm, out_hbm, src_v, dst_v, rows0_v,
          rows1_v, rows2_v, table_sh, agg_sh, sem0, sem1, sem2):
        c = lax.axis_index("c")
        s = lax.axis_index("s")
        wid = c * NSUB + s
        pltpu.async_copy(src_hbm.at[wid], src_v, sem0)
        pltpu.async_copy(dst_hbm.at[wid], dst_v, sem1)
        pltpu.async_copy(table_hbm.at[pl.ds(s * (N // NSUB), N // NSUB)],
                         table_sh.at[pl.ds(s * (N // NSUB), N // NSUB)],
                         sem2)

        def zrow(i, carry):
            for g in range(S // 16):
                rows0_v[i, pl.ds(g * 16, 16)] = jnp.zeros((16,), jnp.float32)
            return carry

        lax.fori_loop(0, CH, zrow, 0)
        pltpu.make_async_copy(src_hbm.at[wid], src_v, sem0).wait()
        pltpu.make_async_copy(dst_hbm.at[wid], dst_v, sem1).wait()
        for q in range(RPT // CH):
            pltpu.async_copy(rows0_v, agg_sh.at[pl.ds(s * RPT + q * CH, CH)],
                             sem0)
        pltpu.make_async_copy(table_hbm.at[pl.ds(s * (N // NSUB), N // NSUB)],
                              table_sh.at[pl.ds(s * (N // NSUB), N // NSUB)],
                              sem2).wait()
        for q in range(RPT // CH):
            pltpu.make_async_copy(rows0_v,
                                  agg_sh.at[pl.ds(s * RPT + q * CH, CH)],
                                  sem0).wait()
        plsc.subcore_barrier()

        banks = (rows0_v, rows1_v, rows2_v)
        sems = (sem0, sem1, sem2)
        pltpu.async_copy(table_sh.at[src_v.at[0]], rows0_v, sem0)
        pltpu.async_copy(table_sh.at[src_v.at[1]], rows1_v, sem1)

        def body(t, carry):
            j0 = 3 * t
            for p in range(3):
                j = j0 + p
                bank = banks[p]
                sem = sems[p]
                bankn = banks[(p + 2) % 3]
                semn = sems[(p + 2) % 3]
                pltpu.make_async_copy(table_sh.at[src_v.at[j]], bank,
                                      sem).wait()
                pltpu.async_copy(bank, agg_sh.at[dst_v.at[j]], sem,
                                 add=True)

                @pl.when(j > 0)
                def _():
                    pltpu.make_async_copy(bankn,
                                          agg_sh.at[dst_v.at[j - 1]],
                                          semn).wait()

                @pl.when(j + 2 < CPW)
                def _():
                    pltpu.async_copy(table_sh.at[src_v.at[j + 2]], bankn,
                                     semn)

            return carry

        lax.fori_loop(0, CPW // 3, body, 0)
        pltpu.make_async_copy(banks[(CPW - 1) % 3],
                              agg_sh.at[dst_v.at[CPW - 1]],
                              sems[(CPW - 1) % 3]).wait()
        plsc.subcore_barrier()

        pltpu.sync_copy(agg_sh.at[pl.ds(s * RPT, RPT)],
                        out_hbm.at[c, pl.ds(s * RPT, RPT)])

    return k(table, srcr, dstr)


def _softmax(z):
    m = jnp.max(z, axis=-1, keepdims=True)
    ez = jnp.exp(z - m)
    return ez / jnp.sum(ez, axis=-1, keepdims=True)


def _enc_body(x_ref, we_ref, be_ref, y0_ref, p0_ref):
    enc = lax.dot_general(x_ref[...], we_ref[...], (((1,), (1,)), ((), ())),
                          preferred_element_type=jnp.float32) + be_ref[...]
    y0 = _softmax(enc)
    y0_ref[...] = y0
    p0_ref[...] = _softmax(y0)


def _tc_encode(x, W_enc, be1):
    return pl.pallas_call(
        _enc_body,
        grid=(GRID,),
        in_specs=[
            pl.BlockSpec((RB, D), lambda i: (i, 0)),
            pl.BlockSpec((S, D), lambda i: (0, 0)),
            pl.BlockSpec((1, S), lambda i: (0, 0)),
        ],
        out_specs=[pl.BlockSpec((RB, S), lambda i: (i, 0))] * 2,
        out_shape=[jax.ShapeDtypeStruct((N, S), jnp.float32)] * 2,
    )(x, W_enc, be1)


def _deg_body(degp_ref, out_ref):
    deg = degp_ref[0, :, 0:1] + degp_ref[1, :, 0:1]
    out_ref[...] = jnp.broadcast_to(jnp.maximum(deg, 1.0), out_ref.shape)


def _tc_degc(degp):
    return pl.pallas_call(
        _deg_body,
        grid=(GRID,),
        in_specs=[pl.BlockSpec((NC, RB, S), lambda i: (0, i, 0))],
        out_specs=pl.BlockSpec((RB, S), lambda i: (i, 0)),
        out_shape=jax.ShapeDtypeStruct((N, S), jnp.float32),
    )(degp)


def _post_body(a_acc, a_y, last, yb_ref, ya_ref, p_ref, agg_ref, deg_ref,
               wc_ref, bc_ref, ya2_ref, p2_ref):
    agg = agg_ref[0] + agg_ref[1]
    aggn = agg / deg_ref[...]
    grad = lax.dot_general(aggn, wc_ref[...], (((1,), (1,)), ((), ())),
                           preferred_element_type=jnp.float32) + bc_ref[...]
    k = jnp.maximum(p_ref[...], EPS) * grad
    k = k - jnp.mean(k, axis=-1, keepdims=True)
    ya2 = ya_ref[...] + a_acc * k
    z = ya2 if last else yb_ref[...] + a_y * k
    ya2_ref[...] = ya2
    p2_ref[...] = _softmax(z)


def _tc_post(yb, ya, p, aggp, degp, W_conv, bc1, stage):
    body = functools.partial(_post_body, A_ACC[stage], A_Y[stage], stage == 3)
    return pl.pallas_call(
        body,
        grid=(GRID,),
        in_specs=[
            pl.BlockSpec((RB, S), lambda i: (i, 0)),
            pl.BlockSpec((RB, S), lambda i: (i, 0)),
            pl.BlockSpec((RB, S), lambda i: (i, 0)),
            pl.BlockSpec((NC, RB, S), lambda i: (0, i, 0)),
            pl.BlockSpec((RB, S), lambda i: (i, 0)),
            pl.BlockSpec((S, S), lambda i: (0, 0)),
            pl.BlockSpec((1, S), lambda i: (0, 0)),
        ],
        out_specs=[pl.BlockSpec((RB, S), lambda i: (i, 0))] * 2,
        out_shape=[jax.ShapeDtypeStruct((N, S), jnp.float32)] * 2,
    )(yb, ya, p, aggp, degp, W_conv, bc1)


def _dec_body(y_ref, wd_ref, bd_ref, out_ref):
    out_ref[...] = lax.dot_general(
        y_ref[...], wd_ref[...], (((1,), (1,)), ((), ())),
        preferred_element_type=jnp.float32) + bd_ref[...]


def _tc_decode(y, W_dec, bd1):
    return pl.pallas_call(
        _dec_body,
        grid=(GRID,),
        in_specs=[
            pl.BlockSpec((RB, S), lambda i: (i, 0)),
            pl.BlockSpec((D, S), lambda i: (0, 0)),
            pl.BlockSpec((1, D), lambda i: (0, 0)),
        ],
        out_specs=pl.BlockSpec((RB, D), lambda i: (i, 0)),
        out_shape=jax.ShapeDtypeStruct((N, D), jnp.float32),
    )(y, W_dec, bd1)


def kernel(x, edge_index, W_enc, b_enc, W_dec, b_dec, W_conv, b_conv):
    src = edge_index[0]
    dst = edge_index[1]
    pad = EPAD - E
    iota = jnp.arange(pad, dtype=jnp.int32)
    srcr = jnp.concatenate([src, iota % N]).reshape(NW, CPW, CH)
    dstr = jnp.concatenate([dst, N + iota % (NOUT - N)]).reshape(NW, CPW, CH)
    be1 = b_enc.reshape(1, S)
    bc1 = b_conv.reshape(1, S)
    bd1 = b_dec.reshape(1, D)

    degc = _tc_degc(_sc_agg(jnp.ones((N, S), jnp.float32), srcr, dstr))
    y0, probs = _tc_encode(x, W_enc, be1)
    yb = y0
    ya = y0
    for _step in range(STEPS):
        for stage in range(4):
            aggp = _sc_agg(probs, srcr, dstr)
            ya, probs = _tc_post(yb, ya, probs, aggp, degc, W_conv, bc1,
                                 stage)
        yb = ya
    return _tc_decode(ya, W_dec, bd1)


# speedup vs baseline: 1.0566x; 1.0226x over previous
import functools

import jax
import jax.numpy as jnp
from jax import lax
from jax.experimental import pallas as pl
from jax.experimental.pallas import tpu as pltpu
from jax.experimental.pallas import tpu_sc as plsc

N = 10000
E = 320000
D = 128
S = 64
STEPS = 8
DT = 0.125
EPS = 1e-12

NC = 2
NSUB = 16
NW = NC * NSUB
CH = 128
CPW = 81
EPAD = NW * CPW * CH
NOUT = 10240
RPT = NOUT // NSUB

RB = 2000
GRID = N // RB

A_ACC = (DT / 6.0, DT / 3.0, DT / 3.0, DT / 6.0)
A_Y = (DT / 2.0, DT / 2.0, DT, 0.0)


def _sc_agg(table, srcr, dstr):
    mesh = plsc.VectorSubcoreMesh(core_axis_name="c", subcore_axis_name="s")

    @functools.partial(
        pl.kernel,
        mesh=mesh,
        out_type=jax.ShapeDtypeStruct((NC, NOUT, S), jnp.float32),
        scratch_types=[
            pltpu.VMEM((CPW, CH), jnp.int32),
            pltpu.VMEM((CPW, CH), jnp.int32),
            pltpu.VMEM((CH, S), jnp.float32),
            pltpu.VMEM((CH, S), jnp.float32),
            pltpu.VMEM((CH, S), jnp.float32),
            pltpu.VMEM_SHARED((N, S), jnp.float32),
            pltpu.VMEM_SHARED((NOUT, S), jnp.float32),
            pltpu.SemaphoreType.DMA,
            pltpu.SemaphoreType.DMA,
            pltpu.SemaphoreType.DMA,
        ],
        compiler_params=pltpu.CompilerParams(use_tc_tiling_on_sc=False),
    )
    def k(table_hbm, src_hbm, dst_hbm, out_hbm, src_v, dst_v, rows0_v,
          rows1_v, rows2_v, table_sh, agg_sh, sem0, sem1, sem2):
        c = lax.axis_index("c")
        s = lax.axis_index("s")
        wid = c * NSUB + s
        pltpu.async_copy(src_hbm.at[wid], src_v, sem0)
        pltpu.async_copy(dst_hbm.at[wid], dst_v, sem1)
        pltpu.async_copy(table_hbm.at[pl.ds(s * (N // NSUB), N // NSUB)],
                         table_sh.at[pl.ds(s * (N // NSUB), N // NSUB)],
                         sem2)

        def zrow(i, carry):
            for g in range(S // 16):
                rows0_v[i, pl.ds(g * 16, 16)] = jnp.zeros((16,), jnp.float32)
            return carry

        lax.fori_loop(0, CH, zrow, 0)
        pltpu.make_async_copy(src_hbm.at[wid], src_v, sem0).wait()
        pltpu.make_async_copy(dst_hbm.at[wid], dst_v, sem1).wait()
        for q in range(RPT // CH):
            pltpu.async_copy(rows0_v, agg_sh.at[pl.ds(s * RPT + q * CH, CH)],
                             sem0)
        pltpu.make_async_copy(table_hbm.at[pl.ds(s * (N // NSUB), N // NSUB)],
                              table_sh.at[pl.ds(s * (N // NSUB), N // NSUB)],
                              sem2).wait()
        for q in range(RPT // CH):
            pltpu.make_async_copy(rows0_v,
                                  agg_sh.at[pl.ds(s * RPT + q * CH, CH)],
                                  sem0).wait()
        plsc.subcore_barrier()

        banks = (rows0_v, rows1_v, rows2_v)
        sems = (sem0, sem1, sem2)
        pltpu.async_copy(table_sh.at[src_v.at[0]], rows0_v, sem0)
        pltpu.async_copy(table_sh.at[src_v.at[1]], rows1_v, sem1)

        def body(t, carry):
            j0 = 3 * t
            for p in range(3):
                j = j0 + p
                bank = banks[p]
                sem = sems[p]
                bankn = banks[(p + 2) % 3]
                semn = sems[(p + 2) % 3]
                pltpu.make_async_copy(table_sh.at[src_v.at[j]], bank,
                                      sem).wait()
                pltpu.async_copy(bank, agg_sh.at[dst_v.at[j]], sem,
                                 add=True)

                @pl.when(j > 0)
                def _():
                    pltpu.make_async_copy(bankn,
                                          agg_sh.at[dst_v.at[j - 1]],
                                          semn).wait()

                @pl.when(j + 2 < CPW)
                def _():
                    pltpu.async_copy(table_sh.at[src_v.at[j + 2]], bankn,
                                     semn)

            return carry

        lax.fori_loop(0, CPW // 3, body, 0)
        pltpu.make_async_copy(banks[(CPW - 1) % 3],
                              agg_sh.at[dst_v.at[CPW - 1]],
                              sems[(CPW - 1) % 3]).wait()
        plsc.subcore_barrier()

        pltpu.sync_copy(agg_sh.at[pl.ds(s * RPT, RPT)],
                        out_hbm.at[c, pl.ds(s * RPT, RPT)])

    return k(table, srcr, dstr)


def _softmax(z):
    m = jnp.max(z, axis=-1, keepdims=True)
    ez = jnp.exp(z - m)
    return ez / jnp.sum(ez, axis=-1, keepdims=True)


def _enc_body(x_ref, we_ref, be_ref, y0_ref, p0_ref):
    enc = lax.dot_general(x_ref[...], we_ref[...], (((1,), (1,)), ((), ())),
                          preferred_element_type=jnp.float32) + be_ref[...]
    y0 = _softmax(enc)
    y0_ref[...] = y0
    p0_ref[...] = _softmax(y0)


def _tc_encode(x, W_enc, be1):
    return pl.pallas_call(
        _enc_body,
        grid=(GRID,),
        in_specs=[
            pl.BlockSpec((RB, D), lambda i: (i, 0)),
            pl.BlockSpec((S, D), lambda i: (0, 0)),
            pl.BlockSpec((1, S), lambda i: (0, 0)),
        ],
        out_specs=[pl.BlockSpec((RB, S), lambda i: (i, 0))] * 2,
        out_shape=[jax.ShapeDtypeStruct((N, S), jnp.float32)] * 2,
    )(x, W_enc, be1)


def _deg_body(degp_ref, out_ref):
    deg = degp_ref[0, :, 0:1] + degp_ref[1, :, 0:1]
    out_ref[...] = jnp.broadcast_to(jnp.maximum(deg, 1.0), out_ref.shape)


def _tc_degc(degp):
    return pl.pallas_call(
        _deg_body,
        grid=(GRID,),
        in_specs=[pl.BlockSpec((NC, RB, S), lambda i: (0, i, 0))],
        out_specs=pl.BlockSpec((RB, S), lambda i: (i, 0)),
        out_shape=jax.ShapeDtypeStruct((N, S), jnp.float32),
    )(degp)


def _post_body(a_acc, a_y, last, yb_ref, ya_ref, p_ref, agg_ref, deg_ref,
               wc_ref, bc_ref, ya2_ref, p2_ref):
    agg = agg_ref[0] + agg_ref[1]
    aggn = agg / deg_ref[...]
    grad = lax.dot_general(aggn, wc_ref[...], (((1,), (1,)), ((), ())),
                           preferred_element_type=jnp.float32) + bc_ref[...]
    k = jnp.maximum(p_ref[...], EPS) * grad
    k = k - jnp.mean(k, axis=-1, keepdims=True)
    ya2 = ya_ref[...] + a_acc * k
    z = ya2 if last else yb_ref[...] + a_y * k
    ya2_ref[...] = ya2
    p2_ref[...] = _softmax(z)


def _tc_post(yb, ya, p, aggp, degp, W_conv, bc1, stage):
    body = functools.partial(_post_body, A_ACC[stage], A_Y[stage], stage == 3)
    return pl.pallas_call(
        body,
        grid=(GRID,),
        in_specs=[
            pl.BlockSpec((RB, S), lambda i: (i, 0)),
            pl.BlockSpec((RB, S), lambda i: (i, 0)),
            pl.BlockSpec((RB, S), lambda i: (i, 0)),
            pl.BlockSpec((NC, RB, S), lambda i: (0, i, 0)),
            pl.BlockSpec((RB, S), lambda i: (i, 0)),
            pl.BlockSpec((S, S), lambda i: (0, 0)),
            pl.BlockSpec((1, S), lambda i: (0, 0)),
        ],
        out_specs=[pl.BlockSpec((RB, S), lambda i: (i, 0))] * 2,
        out_shape=[jax.ShapeDtypeStruct((N, S), jnp.float32)] * 2,
    )(yb, ya, p, aggp, degp, W_conv, bc1)


def _dec_body(y_ref, wd_ref, bd_ref, out_ref):
    out_ref[...] = lax.dot_general(
        y_ref[...], wd_ref[...], (((1,), (1,)), ((), ())),
        preferred_element_type=jnp.float32) + bd_ref[...]


def _tc_decode(y, W_dec, bd1):
    return pl.pallas_call(
        _dec_body,
        grid=(GRID,),
        in_specs=[
            pl.BlockSpec((RB, S), lambda i: (i, 0)),
            pl.BlockSpec((D, S), lambda i: (0, 0)),
            pl.BlockSpec((1, D), lambda i: (0, 0)),
        ],
        out_specs=pl.BlockSpec((RB, D), lambda i: (i, 0)),
        out_shape=jax.ShapeDtypeStruct((N, D), jnp.float32),
    )(y, W_dec, bd1)


def kernel(x, edge_index, W_enc, b_enc, W_dec, b_dec, W_conv, b_conv):
    src = edge_index[0]
    dst = edge_index[1]
    pad = EPAD - E
    iota = jnp.arange(pad, dtype=jnp.int32)
    srcr = jnp.concatenate([src, iota % N]).reshape(NW, CPW, CH)
    dstr = jnp.concatenate([dst, N + iota % (NOUT - N)]).reshape(NW, CPW, CH)
    be1 = b_enc.reshape(1, S)
    bc1 = b_conv.reshape(1, S)
    bd1 = b_dec.reshape(1, D)

    degc = _tc_degc(_sc_agg(jnp.ones((N, S), jnp.float32), srcr, dstr))
    y0, probs = _tc_encode(x, W_enc, be1)
    yb = y0
    ya = y0
    for _step in range(STEPS):
        for stage in range(4):
            aggp = _sc_agg(probs, srcr, dstr)
            ya, probs = _tc_post(yb, ya, probs, aggp, degc, W_conv, bc1,
                                 stage)
        yb = ya
    return _tc_decode(ya, W_dec, bd1)
